# baseline (speedup 1.0000x reference)
import functools

import jax
import jax.numpy as jnp
from jax.experimental import pallas as pl

_N = 50000
_A = 512
_BLK = 1000


def _residual_linear_kernel(x_ref, w_ref, b_ref, o_ref):
    xb = x_ref[...]
    acc = jax.lax.dot_general(
        xb, w_ref[...],
        dimension_numbers=(((1,), (1,)), ((), ())),
        preferred_element_type=jnp.float32,
    )
    o_ref[...] = xb + 0.1 * acc + 0.1 * b_ref[...]


@functools.partial(jax.jit, static_argnames=())
def kernel(x, W, b):
    b2 = b.reshape(1, _A)
    grid = (_N // _BLK,)
    return pl.pallas_call(
        _residual_linear_kernel,
        grid=grid,
        in_specs=[
            pl.BlockSpec((_BLK, _A), lambda i: (i, 0)),
            pl.BlockSpec((_A, _A), lambda i: (0, 0)),
            pl.BlockSpec((1, _A), lambda i: (0, 0)),
        ],
        out_specs=pl.BlockSpec((_BLK, _A), lambda i: (i, 0)),
        out_shape=jax.ShapeDtypeStruct((_N, _A), jnp.float32),
    )(x, W, b2)


# speedup vs baseline: 1.2730x; 1.2730x over previous
import functools

import jax
import jax.numpy as jnp
from jax.experimental import pallas as pl

_N = 50000
_A = 512
_BLK = 5000


def _residual_linear_kernel(x_ref, w_ref, b_ref, o_ref):
    xb = x_ref[...]
    acc = jax.lax.dot_general(
        xb, w_ref[...],
        dimension_numbers=(((1,), (1,)), ((), ())),
        preferred_element_type=jnp.float32,
    )
    o_ref[...] = xb + 0.1 * acc + 0.1 * b_ref[...]


@functools.partial(jax.jit, static_argnames=())
def kernel(x, W, b):
    b2 = b.reshape(1, _A)
    grid = (_N // _BLK,)
    return pl.pallas_call(
        _residual_linear_kernel,
        grid=grid,
        in_specs=[
            pl.BlockSpec((_BLK, _A), lambda i: (i, 0)),
            pl.BlockSpec((_A, _A), lambda i: (0, 0)),
            pl.BlockSpec((1, _A), lambda i: (0, 0)),
        ],
        out_specs=pl.BlockSpec((_BLK, _A), lambda i: (i, 0)),
        out_shape=jax.ShapeDtypeStruct((_N, _A), jnp.float32),
    )(x, W, b2)
